# baseline (speedup 1.0000x reference)
import functools

import jax
import jax.numpy as jnp
from jax import lax
from jax.experimental import pallas as pl
from jax.experimental.pallas import tpu as pltpu
from jax.experimental.pallas import tpu_sc as plsc

BATCH = 16384
NFIELD = 2
EMBED = 64
FLAT = BATCH * NFIELD
FIELD_OFFSET = 1000000
BN_EPS = 1e-5

_info = plsc.get_sparse_core_info()
_NC, _NS = _info.num_cores, _info.num_subcores
_NW = _NC * _NS
_BPW = FLAT // _NW
_EB = 32
_NB = _BPW // _EB
_GPB = _EB * EMBED
_NSTR = _GPB // 128
_OUTR = _BPW // NFIELD
_GPF = NFIELD * FIELD_OFFSET // 16


def _gather_body(tab_hbm, idx_hbm, out_hbm, idx_v, gidx_v, buf_v, out_v, sem):
    wid = lax.axis_index("s") * _NC + lax.axis_index("c")
    pltpu.sync_copy(idx_hbm.at[wid], idx_v)
    lanes = lax.iota(jnp.int32, 16)
    cq = [(q * 16 + lanes) * _GPF for q in range(EMBED // 16)]

    def batch(b, carry):
        base = b * _EB
        vv = [idx_v[pl.ds(base + 16 * h, 16)] for h in range(_EB // 16)]
        ls = []
        for e in range(_EB):
            v = vv[e // 16][e % 16] + (e % 2) * FIELD_OFFSET
            ls.append(v & 15)
            evec = jnp.full((16,), v >> 4, jnp.int32)
            for q in range(EMBED // 16):
                gidx_v[e >> 1, pl.ds((e % 2) * 64 + q * 16, 16)] = evec + cq[q]
        copies = []
        for r in range(_NSTR):
            copies.append(
                pltpu.async_copy(
                    tab_hbm.at[gidx_v.at[r]],
                    buf_v.at[pl.ds(r * 128, 128)],
                    sem,
                )
            )
        for c in copies:
            c.wait()
        for e in range(_EB):
            k = base + e
            row = k >> 1
            col = (e % 2) * EMBED
            lvec = jnp.full((16,), ls[e], jnp.int32)
            for q in range(EMBED // 16):
                vals = plsc.load_gather(
                    buf_v, [e * 64 + q * 16 + lanes, lvec]
                )
                out_v[row, pl.ds(col + q * 16, 16)] = vals
        return carry

    lax.fori_loop(0, _NB, batch, 0)
    pltpu.sync_copy(out_v, out_hbm.at[pl.ds(wid * _OUTR, _OUTR)])


_gather = functools.partial(
    pl.kernel,
    out_type=jax.ShapeDtypeStruct((BATCH, NFIELD * EMBED), jnp.float32),
    mesh=plsc.VectorSubcoreMesh(core_axis_name="c", subcore_axis_name="s"),
    scratch_types=[
        pltpu.VMEM((_BPW,), jnp.int32),
        pltpu.VMEM((_NSTR, 128), jnp.int32),
        pltpu.VMEM((_GPB, 16), jnp.float32),
        pltpu.VMEM((_OUTR, NFIELD * EMBED), jnp.float32),
        pltpu.SemaphoreType.DMA,
    ],
    compiler_params=pltpu.CompilerParams(
        use_tc_tiling_on_sc=False, needs_layout_passes=False
    ),
)(_gather_body)


def _mlp_body(e_ref, w0, c0, w1, c1, w2, c2, wo, co, out_ref):
    h = e_ref[...]
    h = jnp.maximum(jnp.dot(h, w0[...], preferred_element_type=jnp.float32) + c0[...], 0.0)
    h = jnp.maximum(jnp.dot(h, w1[...], preferred_element_type=jnp.float32) + c1[...], 0.0)
    h = jnp.maximum(jnp.dot(h, w2[...], preferred_element_type=jnp.float32) + c2[...], 0.0)
    out_ref[...] = jnp.maximum(
        jnp.dot(h, wo[...], preferred_element_type=jnp.float32) + co[...], 0.0
    )


_MLP_BLK = 2048


def _mlp(e, w0, c0, w1, c1, w2, c2, wo, co):
    din = NFIELD * EMBED
    full = lambda shape: pl.BlockSpec(shape, lambda i: (0, 0))
    return pl.pallas_call(
        _mlp_body,
        grid=(BATCH // _MLP_BLK,),
        in_specs=[
            pl.BlockSpec((_MLP_BLK, din), lambda i: (i, 0)),
            full(w0.shape), full(c0.shape),
            full(w1.shape), full(c1.shape),
            full(w2.shape), full(c2.shape),
            full(wo.shape), full(co.shape),
        ],
        out_specs=pl.BlockSpec((_MLP_BLK, 1), lambda i: (i, 0)),
        out_shape=jax.ShapeDtypeStruct((BATCH, 1), jnp.float32),
    )(e, w0, c0, w1, c1, w2, c2, wo, co)


def kernel(x, emb, W0, b0, g0, be0, W1, b1, g1, be1, W2, b2, g2, be2, Wo, bo):
    idx = x.astype(jnp.int32).reshape(_NW, _BPW)
    tab = emb.T.reshape(EMBED * _GPF, 16)
    e = _gather(tab, idx)

    inv = 1.0 / jnp.sqrt(jnp.float32(1.0 + BN_EPS))
    s0, s1, s2 = g0 * inv, g1 * inv, g2 * inv
    w0 = W0 * s0[None, :]
    c0 = (b0 * s0 + be0).reshape(1, -1)
    w1 = W1 * s1[None, :]
    c1 = (b1 * s1 + be1).reshape(1, -1)
    w2 = W2 * s2[None, :]
    c2 = (b2 * s2 + be2).reshape(1, -1)
    co = bo.reshape(1, 1)

    return _mlp(e, w0, c0, w1, c1, w2, c2, Wo, co)


# speedup vs baseline: 1.0004x; 1.0004x over previous
import functools

import jax
import jax.numpy as jnp
from jax import lax
from jax.experimental import pallas as pl
from jax.experimental.pallas import tpu as pltpu
from jax.experimental.pallas import tpu_sc as plsc

BATCH = 16384
NFIELD = 2
EMBED = 64
FLAT = BATCH * NFIELD
FIELD_OFFSET = 1000000
BN_EPS = 1e-5

_info = plsc.get_sparse_core_info()
_NC, _NS = _info.num_cores, _info.num_subcores
_NW = _NC * _NS
_BPW = FLAT // _NW
_EB = 32
_NB = _BPW // _EB
_GPB = _EB * EMBED
_NSTR = _GPB // 128
_OUTR = _BPW // NFIELD
_GPF = NFIELD * FIELD_OFFSET // 16


def _gather_body(tab_hbm, idx_hbm, out_hbm, idx_v, gidx_v, buf_v, out_v, sem):
    wid = lax.axis_index("s") * _NC + lax.axis_index("c")
    pltpu.sync_copy(idx_hbm.at[wid], idx_v)
    lanes = lax.iota(jnp.int32, 16)
    cq = [(q * 16 + lanes) * _GPF for q in range(EMBED // 16)]

    @pl.loop(0, _NB)
    def batch(b):
        base = b * _EB
        vv = [idx_v[pl.ds(base + 16 * h, 16)] for h in range(_EB // 16)]
        ls = []
        for e in range(_EB):
            v = vv[e // 16][e % 16] + (e % 2) * FIELD_OFFSET
            ls.append(v & 15)
            evec = jnp.full((16,), v >> 4, jnp.int32)
            for q in range(EMBED // 16):
                gidx_v[e >> 1, pl.ds((e % 2) * 64 + q * 16, 16)] = evec + cq[q]
        copies = []
        for r in range(_NSTR):
            copies.append(
                pltpu.async_copy(
                    tab_hbm.at[gidx_v.at[r]],
                    buf_v.at[pl.ds(r * 128, 128)],
                    sem,
                )
            )
        for c in copies:
            c.wait()
        for e in range(_EB):
            k = base + e
            row = k >> 1
            col = (e % 2) * EMBED
            lvec = jnp.full((16,), ls[e], jnp.int32)
            for q in range(EMBED // 16):
                vals = plsc.load_gather(
                    buf_v, [e * 64 + q * 16 + lanes, lvec]
                )
                out_v[row, pl.ds(col + q * 16, 16)] = vals

    pltpu.sync_copy(out_v, out_hbm.at[pl.ds(wid * _OUTR, _OUTR)])


_gather = functools.partial(
    pl.kernel,
    out_type=jax.ShapeDtypeStruct((BATCH, NFIELD * EMBED), jnp.float32),
    mesh=plsc.VectorSubcoreMesh(core_axis_name="c", subcore_axis_name="s"),
    scratch_types=[
        pltpu.VMEM((_BPW,), jnp.int32),
        pltpu.VMEM((_NSTR, 128), jnp.int32),
        pltpu.VMEM((_GPB, 16), jnp.float32),
        pltpu.VMEM((_OUTR, NFIELD * EMBED), jnp.float32),
        pltpu.SemaphoreType.DMA,
    ],
    compiler_params=pltpu.CompilerParams(
        use_tc_tiling_on_sc=False, needs_layout_passes=False
    ),
)(_gather_body)


def _mlp_body(e_ref, w0, c0, w1, c1, w2, c2, wo, co, out_ref):
    h = e_ref[...]
    h = jnp.maximum(jnp.dot(h, w0[...], preferred_element_type=jnp.float32) + c0[...], 0.0)
    h = jnp.maximum(jnp.dot(h, w1[...], preferred_element_type=jnp.float32) + c1[...], 0.0)
    h = jnp.maximum(jnp.dot(h, w2[...], preferred_element_type=jnp.float32) + c2[...], 0.0)
    out_ref[...] = jnp.maximum(
        jnp.dot(h, wo[...], preferred_element_type=jnp.float32) + co[...], 0.0
    )


_MLP_BLK = 2048


def _mlp(e, w0, c0, w1, c1, w2, c2, wo, co):
    din = NFIELD * EMBED
    full = lambda shape: pl.BlockSpec(shape, lambda i: (0, 0))
    return pl.pallas_call(
        _mlp_body,
        grid=(BATCH // _MLP_BLK,),
        in_specs=[
            pl.BlockSpec((_MLP_BLK, din), lambda i: (i, 0)),
            full(w0.shape), full(c0.shape),
            full(w1.shape), full(c1.shape),
            full(w2.shape), full(c2.shape),
            full(wo.shape), full(co.shape),
        ],
        out_specs=pl.BlockSpec((_MLP_BLK, 1), lambda i: (i, 0)),
        out_shape=jax.ShapeDtypeStruct((BATCH, 1), jnp.float32),
    )(e, w0, c0, w1, c1, w2, c2, wo, co)


def kernel(x, emb, W0, b0, g0, be0, W1, b1, g1, be1, W2, b2, g2, be2, Wo, bo):
    idx = x.astype(jnp.int32).reshape(_NW, _BPW)
    tab = emb.T.reshape(EMBED * _GPF, 16)
    e = _gather(tab, idx)

    inv = 1.0 / jnp.sqrt(jnp.float32(1.0 + BN_EPS))
    s0, s1, s2 = g0 * inv, g1 * inv, g2 * inv
    w0 = W0 * s0[None, :]
    c0 = (b0 * s0 + be0).reshape(1, -1)
    w1 = W1 * s1[None, :]
    c1 = (b1 * s1 + be1).reshape(1, -1)
    w2 = W2 * s2[None, :]
    c2 = (b2 * s2 + be2).reshape(1, -1)
    co = bo.reshape(1, 1)

    return _mlp(e, w0, c0, w1, c1, w2, c2, Wo, co)


# speedup vs baseline: 16.4592x; 16.4532x over previous
import functools

import jax
import jax.numpy as jnp
from jax import lax
from jax.experimental import pallas as pl
from jax.experimental.pallas import tpu as pltpu
from jax.experimental.pallas import tpu_sc as plsc

BATCH = 16384
NFIELD = 2
EMBED = 64
FLAT = BATCH * NFIELD
FIELD_OFFSET = 1000000
BN_EPS = 1e-5

_info = plsc.get_sparse_core_info()
_NC, _NS = _info.num_cores, _info.num_subcores
_NW = _NC * _NS
_BPW = FLAT // _NW
_EB = 32
_NB = _BPW // _EB
_GPB = _EB * EMBED
_NSTR = _GPB // 128
_OUTR = _BPW // NFIELD
_GPF = NFIELD * FIELD_OFFSET // 16


_PSEG = 640
_NPIECE = NFIELD * FIELD_OFFSET // _PSEG


def _launder_body(embt_hbm, out_hbm, buf_v, sem):
    wid = lax.axis_index("s") * _NC + lax.axis_index("c")

    @pl.loop(0, (_NPIECE + _NW - 1) // _NW)
    def it(i):
        t = wid + i * _NW

        @pl.when(t < _NPIECE)
        def _():
            c = t * _PSEG
            pltpu.sync_copy(embt_hbm.at[:, pl.ds(c, _PSEG)], buf_v)
            copies = []
            for f in range(EMBED):
                copies.append(
                    pltpu.async_copy(
                        buf_v.at[f],
                        out_hbm.at[pl.ds(f * NFIELD * FIELD_OFFSET + c, _PSEG)],
                        sem,
                    )
                )
            for cp in copies:
                cp.wait()


_launder = functools.partial(
    pl.kernel,
    out_type=jax.ShapeDtypeStruct((NFIELD * FIELD_OFFSET * EMBED,), jnp.float32),
    mesh=plsc.VectorSubcoreMesh(core_axis_name="c", subcore_axis_name="s"),
    scratch_types=[
        pltpu.VMEM((EMBED, _PSEG), jnp.float32),
        pltpu.SemaphoreType.DMA,
    ],
)(_launder_body)


def _gather_body(tab_hbm, idx_hbm, out_hbm, idx_v, gidx_v, buf_v, out_v, sem):
    wid = lax.axis_index("s") * _NC + lax.axis_index("c")
    pltpu.sync_copy(idx_hbm.at[wid], idx_v)
    lanes = lax.iota(jnp.int32, 16)
    cq = [(q * 16 + lanes) * _GPF for q in range(EMBED // 16)]

    @pl.loop(0, _NB)
    def batch(b):
        base = b * _EB
        vv = [idx_v[pl.ds(base + 16 * h, 16)] for h in range(_EB // 16)]
        ls = []
        for e in range(_EB):
            v = vv[e // 16][e % 16] + (e % 2) * FIELD_OFFSET
            ls.append(v & 15)
            evec = jnp.full((16,), v >> 4, jnp.int32)
            for q in range(EMBED // 16):
                gidx_v[e >> 1, pl.ds((e % 2) * 64 + q * 16, 16)] = evec + cq[q]
        copies = []
        for r in range(_NSTR):
            copies.append(
                pltpu.async_copy(
                    tab_hbm.at[gidx_v.at[r]],
                    buf_v.at[pl.ds(r * 128, 128)],
                    sem,
                )
            )
        for c in copies:
            c.wait()
        for e in range(_EB):
            k = base + e
            row = k >> 1
            col = (e % 2) * EMBED
            lvec = jnp.full((16,), ls[e], jnp.int32)
            for q in range(EMBED // 16):
                vals = plsc.load_gather(
                    buf_v, [e * 64 + q * 16 + lanes, lvec]
                )
                out_v[row, pl.ds(col + q * 16, 16)] = vals

    pltpu.sync_copy(out_v, out_hbm.at[pl.ds(wid * _OUTR, _OUTR)])


_gather = functools.partial(
    pl.kernel,
    out_type=jax.ShapeDtypeStruct((BATCH, NFIELD * EMBED), jnp.float32),
    mesh=plsc.VectorSubcoreMesh(core_axis_name="c", subcore_axis_name="s"),
    scratch_types=[
        pltpu.VMEM((_BPW,), jnp.int32),
        pltpu.VMEM((_NSTR, 128), jnp.int32),
        pltpu.VMEM((_GPB, 16), jnp.float32),
        pltpu.VMEM((_OUTR, NFIELD * EMBED), jnp.float32),
        pltpu.SemaphoreType.DMA,
    ],
    compiler_params=pltpu.CompilerParams(
        use_tc_tiling_on_sc=False, needs_layout_passes=False
    ),
)(_gather_body)


def _mlp_body(e_ref, w0, c0, w1, c1, w2, c2, wo, co, out_ref):
    h = e_ref[...]
    h = jnp.maximum(jnp.dot(h, w0[...], preferred_element_type=jnp.float32) + c0[...], 0.0)
    h = jnp.maximum(jnp.dot(h, w1[...], preferred_element_type=jnp.float32) + c1[...], 0.0)
    h = jnp.maximum(jnp.dot(h, w2[...], preferred_element_type=jnp.float32) + c2[...], 0.0)
    out_ref[...] = jnp.maximum(
        jnp.dot(h, wo[...], preferred_element_type=jnp.float32) + co[...], 0.0
    )


_MLP_BLK = 2048


def _mlp(e, w0, c0, w1, c1, w2, c2, wo, co):
    din = NFIELD * EMBED
    full = lambda shape: pl.BlockSpec(shape, lambda i: (0, 0))
    return pl.pallas_call(
        _mlp_body,
        grid=(BATCH // _MLP_BLK,),
        in_specs=[
            pl.BlockSpec((_MLP_BLK, din), lambda i: (i, 0)),
            full(w0.shape), full(c0.shape),
            full(w1.shape), full(c1.shape),
            full(w2.shape), full(c2.shape),
            full(wo.shape), full(co.shape),
        ],
        out_specs=pl.BlockSpec((_MLP_BLK, 1), lambda i: (i, 0)),
        out_shape=jax.ShapeDtypeStruct((BATCH, 1), jnp.float32),
    )(e, w0, c0, w1, c1, w2, c2, wo, co)


def kernel(x, emb, W0, b0, g0, be0, W1, b1, g1, be1, W2, b2, g2, be2, Wo, bo):
    idx = x.astype(jnp.int32).reshape(_NW, _BPW)
    flat = _launder(emb.T)
    tab = flat.reshape(EMBED * _GPF, 16)
    e = _gather(tab, idx)

    inv = 1.0 / jnp.sqrt(jnp.float32(1.0 + BN_EPS))
    s0, s1, s2 = g0 * inv, g1 * inv, g2 * inv
    w0 = W0 * s0[None, :]
    c0 = (b0 * s0 + be0).reshape(1, -1)
    w1 = W1 * s1[None, :]
    c1 = (b1 * s1 + be1).reshape(1, -1)
    w2 = W2 * s2[None, :]
    c2 = (b2 * s2 + be2).reshape(1, -1)
    co = bo.reshape(1, 1)

    return _mlp(e, w0, c0, w1, c1, w2, c2, Wo, co)


# speedup vs baseline: 18.1486x; 1.1026x over previous
import functools

import jax
import jax.numpy as jnp
from jax import lax
from jax.experimental import pallas as pl
from jax.experimental.pallas import tpu as pltpu
from jax.experimental.pallas import tpu_sc as plsc

BATCH = 16384
NFIELD = 2
EMBED = 64
FLAT = BATCH * NFIELD
FIELD_OFFSET = 1000000
BN_EPS = 1e-5

_info = plsc.get_sparse_core_info()
_NC, _NS = _info.num_cores, _info.num_subcores
_NW = _NC * _NS
_BPW = FLAT // _NW
_EB = 32
_NB = _BPW // _EB
_GPB = _EB * EMBED
_NSTR = _GPB // 128
_OUTR = _BPW // NFIELD
_GPF = NFIELD * FIELD_OFFSET // 16


_PSEG = 640
_NPIECE = NFIELD * FIELD_OFFSET // _PSEG


def _launder_body(embt_hbm, out_hbm, buf0_v, buf1_v, rsem, wsem):
    wid = lax.axis_index("s") * _NC + lax.axis_index("c")
    bufs = (buf0_v, buf1_v)
    pltpu.async_copy(embt_hbm.at[:, pl.ds(wid * _PSEG, _PSEG)], buf0_v, rsem)

    @pl.loop(0, (_NPIECE + 2 * _NW - 1) // (2 * _NW))
    def it(i2):
        for j in range(2):
            t = wid + (2 * i2 + j) * _NW

            @pl.when(t < _NPIECE)
            def _(t=t, j=j):
                c = t * _PSEG
                pltpu.make_async_copy(
                    embt_hbm.at[:, pl.ds(c, _PSEG)], bufs[j], rsem
                ).wait()
                tn = t + _NW

                @pl.when(tn < _NPIECE)
                def __():
                    pltpu.async_copy(
                        embt_hbm.at[:, pl.ds(tn * _PSEG, _PSEG)], bufs[1 - j], rsem
                    )

                copies = []
                for f in range(EMBED):
                    copies.append(
                        pltpu.async_copy(
                            bufs[j].at[f],
                            out_hbm.at[pl.ds(f * NFIELD * FIELD_OFFSET + c, _PSEG)],
                            wsem,
                        )
                    )
                for cp in copies:
                    cp.wait()


_launder = functools.partial(
    pl.kernel,
    out_type=jax.ShapeDtypeStruct((NFIELD * FIELD_OFFSET * EMBED,), jnp.float32),
    mesh=plsc.VectorSubcoreMesh(core_axis_name="c", subcore_axis_name="s"),
    scratch_types=[
        pltpu.VMEM((EMBED, _PSEG), jnp.float32),
        pltpu.VMEM((EMBED, _PSEG), jnp.float32),
        pltpu.SemaphoreType.DMA,
        pltpu.SemaphoreType.DMA,
    ],
)(_launder_body)


def _gather_body(tab_hbm, idx_hbm, out_hbm, idx_v, gidx_v, buf_v, out_v, sem):
    wid = lax.axis_index("s") * _NC + lax.axis_index("c")
    pltpu.sync_copy(idx_hbm.at[wid], idx_v)
    lanes = lax.iota(jnp.int32, 16)
    cq = [(q * 16 + lanes) * _GPF for q in range(EMBED // 16)]

    @pl.loop(0, _NB)
    def batch(b):
        base = b * _EB
        vv = [idx_v[pl.ds(base + 16 * h, 16)] for h in range(_EB // 16)]
        ls = []
        for e in range(_EB):
            v = vv[e // 16][e % 16] + (e % 2) * FIELD_OFFSET
            ls.append(v & 15)
            evec = jnp.full((16,), v >> 4, jnp.int32)
            for q in range(EMBED // 16):
                gidx_v[e >> 1, pl.ds((e % 2) * 64 + q * 16, 16)] = evec + cq[q]
        copies = []
        for r in range(_NSTR):
            copies.append(
                pltpu.async_copy(
                    tab_hbm.at[gidx_v.at[r]],
                    buf_v.at[pl.ds(r * 128, 128)],
                    sem,
                )
            )
        for c in copies:
            c.wait()
        for e in range(_EB):
            k = base + e
            row = k >> 1
            col = (e % 2) * EMBED
            lvec = jnp.full((16,), ls[e], jnp.int32)
            for q in range(EMBED // 16):
                vals = plsc.load_gather(
                    buf_v, [e * 64 + q * 16 + lanes, lvec]
                )
                out_v[row, pl.ds(col + q * 16, 16)] = vals

    pltpu.sync_copy(out_v, out_hbm.at[pl.ds(wid * _OUTR, _OUTR)])


_gather = functools.partial(
    pl.kernel,
    out_type=jax.ShapeDtypeStruct((BATCH, NFIELD * EMBED), jnp.float32),
    mesh=plsc.VectorSubcoreMesh(core_axis_name="c", subcore_axis_name="s"),
    scratch_types=[
        pltpu.VMEM((_BPW,), jnp.int32),
        pltpu.VMEM((_NSTR, 128), jnp.int32),
        pltpu.VMEM((_GPB, 16), jnp.float32),
        pltpu.VMEM((_OUTR, NFIELD * EMBED), jnp.float32),
        pltpu.SemaphoreType.DMA,
    ],
    compiler_params=pltpu.CompilerParams(
        use_tc_tiling_on_sc=False, needs_layout_passes=False
    ),
)(_gather_body)


def _mlp_body(e_ref, w0, c0, w1, c1, w2, c2, wo, co, out_ref):
    h = e_ref[...]
    h = jnp.maximum(jnp.dot(h, w0[...], preferred_element_type=jnp.float32) + c0[...], 0.0)
    h = jnp.maximum(jnp.dot(h, w1[...], preferred_element_type=jnp.float32) + c1[...], 0.0)
    h = jnp.maximum(jnp.dot(h, w2[...], preferred_element_type=jnp.float32) + c2[...], 0.0)
    out_ref[...] = jnp.maximum(
        jnp.dot(h, wo[...], preferred_element_type=jnp.float32) + co[...], 0.0
    )


_MLP_BLK = 2048


def _mlp(e, w0, c0, w1, c1, w2, c2, wo, co):
    din = NFIELD * EMBED
    full = lambda shape: pl.BlockSpec(shape, lambda i: (0, 0))
    return pl.pallas_call(
        _mlp_body,
        grid=(BATCH // _MLP_BLK,),
        in_specs=[
            pl.BlockSpec((_MLP_BLK, din), lambda i: (i, 0)),
            full(w0.shape), full(c0.shape),
            full(w1.shape), full(c1.shape),
            full(w2.shape), full(c2.shape),
            full(wo.shape), full(co.shape),
        ],
        out_specs=pl.BlockSpec((_MLP_BLK, 1), lambda i: (i, 0)),
        out_shape=jax.ShapeDtypeStruct((BATCH, 1), jnp.float32),
    )(e, w0, c0, w1, c1, w2, c2, wo, co)


def kernel(x, emb, W0, b0, g0, be0, W1, b1, g1, be1, W2, b2, g2, be2, Wo, bo):
    idx = x.astype(jnp.int32).reshape(_NW, _BPW)
    flat = _launder(emb.T)
    tab = flat.reshape(EMBED * _GPF, 16)
    e = _gather(tab, idx)

    inv = 1.0 / jnp.sqrt(jnp.float32(1.0 + BN_EPS))
    s0, s1, s2 = g0 * inv, g1 * inv, g2 * inv
    w0 = W0 * s0[None, :]
    c0 = (b0 * s0 + be0).reshape(1, -1)
    w1 = W1 * s1[None, :]
    c1 = (b1 * s1 + be1).reshape(1, -1)
    w2 = W2 * s2[None, :]
    c2 = (b2 * s2 + be2).reshape(1, -1)
    co = bo.reshape(1, 1)

    return _mlp(e, w0, c0, w1, c1, w2, c2, Wo, co)


# speedup vs baseline: 19.1181x; 1.0534x over previous
import functools

import jax
import jax.numpy as jnp
from jax import lax
from jax.experimental import pallas as pl
from jax.experimental.pallas import tpu as pltpu
from jax.experimental.pallas import tpu_sc as plsc

BATCH = 16384
NFIELD = 2
EMBED = 64
FLAT = BATCH * NFIELD
FIELD_OFFSET = 1000000
BN_EPS = 1e-5

_info = plsc.get_sparse_core_info()
_NC, _NS = _info.num_cores, _info.num_subcores
_NW = _NC * _NS
_BPW = FLAT // _NW
_EB = 16
_NB = _BPW // _EB
_GPB = _EB * EMBED
_NSTR = _GPB // 128
_OUTR = _BPW // NFIELD
_GPF = NFIELD * FIELD_OFFSET // 16


_PSEG = 640
_NPIECE = NFIELD * FIELD_OFFSET // _PSEG


def _launder_body(embt_hbm, out_hbm, buf0_v, buf1_v, rsem, wsem):
    wid = lax.axis_index("s") * _NC + lax.axis_index("c")
    bufs = (buf0_v, buf1_v)
    pltpu.async_copy(embt_hbm.at[:, pl.ds(wid * _PSEG, _PSEG)], buf0_v, rsem)

    @pl.loop(0, (_NPIECE + 2 * _NW - 1) // (2 * _NW))
    def it(i2):
        for j in range(2):
            t = wid + (2 * i2 + j) * _NW

            @pl.when(t < _NPIECE)
            def _(t=t, j=j):
                c = t * _PSEG
                pltpu.make_async_copy(
                    embt_hbm.at[:, pl.ds(c, _PSEG)], bufs[j], rsem
                ).wait()
                tn = t + _NW

                @pl.when(tn < _NPIECE)
                def __():
                    pltpu.async_copy(
                        embt_hbm.at[:, pl.ds(tn * _PSEG, _PSEG)], bufs[1 - j], rsem
                    )

                copies = []
                for f in range(EMBED):
                    copies.append(
                        pltpu.async_copy(
                            bufs[j].at[f],
                            out_hbm.at[pl.ds(f * NFIELD * FIELD_OFFSET + c, _PSEG)],
                            wsem,
                        )
                    )
                for cp in copies:
                    cp.wait()


_launder = functools.partial(
    pl.kernel,
    out_type=jax.ShapeDtypeStruct((NFIELD * FIELD_OFFSET * EMBED,), jnp.float32),
    mesh=plsc.VectorSubcoreMesh(core_axis_name="c", subcore_axis_name="s"),
    scratch_types=[
        pltpu.VMEM((EMBED, _PSEG), jnp.float32),
        pltpu.VMEM((EMBED, _PSEG), jnp.float32),
        pltpu.SemaphoreType.DMA,
        pltpu.SemaphoreType.DMA,
    ],
)(_launder_body)


def _gather_body(tab_hbm, idx_hbm, out_hbm, idx_v, g0_v, g1_v, b0_v, b1_v, out_v, sem):
    wid = lax.axis_index("s") * _NC + lax.axis_index("c")
    pltpu.sync_copy(idx_hbm.at[wid], idx_v)
    lanes = lax.iota(jnp.int32, 16)
    cq = [(q * 16 + lanes) * _GPF for q in range(EMBED // 16)]
    gidx = (g0_v, g1_v)
    bufs = (b0_v, b1_v)

    def build_and_fire(b, j):
        base = b * _EB
        vv = [idx_v[pl.ds(base + 16 * h, 16)] for h in range(_EB // 16)]
        for e in range(_EB):
            v = vv[e // 16][e % 16] + (e % 2) * FIELD_OFFSET
            evec = jnp.full((16,), v >> 4, jnp.int32)
            for q in range(EMBED // 16):
                gidx[j][(e * EMBED + q * 16) // 128,
                        pl.ds((e * EMBED + q * 16) % 128, 16)] = evec + cq[q]
        for r in range(_NSTR):
            pltpu.async_copy(
                tab_hbm.at[gidx[j].at[r]],
                bufs[j].at[pl.ds(r * 128, 128)],
                sem,
            )

    def drain_and_extract(b, j):
        for r in range(_NSTR):
            pltpu.make_async_copy(
                tab_hbm.at[gidx[j].at[r]],
                bufs[j].at[pl.ds(r * 128, 128)],
                sem,
            ).wait()
        base = b * _EB
        vv = [idx_v[pl.ds(base + 16 * h, 16)] for h in range(_EB // 16)]
        for e in range(_EB):
            v = vv[e // 16][e % 16] + (e % 2) * FIELD_OFFSET
            k = base + e
            row = k >> 1
            col = (e % 2) * EMBED
            lvec = jnp.full((16,), v & 15, jnp.int32)
            for q in range(EMBED // 16):
                vals = plsc.load_gather(
                    bufs[j], [e * 64 + q * 16 + lanes, lvec]
                )
                out_v[row, pl.ds(col + q * 16, 16)] = vals

    build_and_fire(0, 0)

    @pl.loop(0, _NB // 2)
    def batch(i2):
        for j in range(2):
            b = i2 * 2 + j

            @pl.when(b + 1 < _NB)
            def _(b=b, j=j):
                build_and_fire(b + 1, 1 - j)

            drain_and_extract(b, j)

    pltpu.sync_copy(out_v, out_hbm.at[pl.ds(wid * _OUTR, _OUTR)])


_gather = functools.partial(
    pl.kernel,
    out_type=jax.ShapeDtypeStruct((BATCH, NFIELD * EMBED), jnp.float32),
    mesh=plsc.VectorSubcoreMesh(core_axis_name="c", subcore_axis_name="s"),
    scratch_types=[
        pltpu.VMEM((_BPW,), jnp.int32),
        pltpu.VMEM((_NSTR, 128), jnp.int32),
        pltpu.VMEM((_NSTR, 128), jnp.int32),
        pltpu.VMEM((_GPB, 16), jnp.float32),
        pltpu.VMEM((_GPB, 16), jnp.float32),
        pltpu.VMEM((_OUTR, NFIELD * EMBED), jnp.float32),
        pltpu.SemaphoreType.DMA,
    ],
    compiler_params=pltpu.CompilerParams(
        use_tc_tiling_on_sc=False, needs_layout_passes=False
    ),
)(_gather_body)


def _mlp_body(e_ref, w0, c0, w1, c1, w2, c2, wo, co, out_ref):
    h = e_ref[...]
    h = jnp.maximum(jnp.dot(h, w0[...], preferred_element_type=jnp.float32) + c0[...], 0.0)
    h = jnp.maximum(jnp.dot(h, w1[...], preferred_element_type=jnp.float32) + c1[...], 0.0)
    h = jnp.maximum(jnp.dot(h, w2[...], preferred_element_type=jnp.float32) + c2[...], 0.0)
    out_ref[...] = jnp.maximum(
        jnp.dot(h, wo[...], preferred_element_type=jnp.float32) + co[...], 0.0
    )


_MLP_BLK = 2048


def _mlp(e, w0, c0, w1, c1, w2, c2, wo, co):
    din = NFIELD * EMBED
    full = lambda shape: pl.BlockSpec(shape, lambda i: (0, 0))
    return pl.pallas_call(
        _mlp_body,
        grid=(BATCH // _MLP_BLK,),
        in_specs=[
            pl.BlockSpec((_MLP_BLK, din), lambda i: (i, 0)),
            full(w0.shape), full(c0.shape),
            full(w1.shape), full(c1.shape),
            full(w2.shape), full(c2.shape),
            full(wo.shape), full(co.shape),
        ],
        out_specs=pl.BlockSpec((_MLP_BLK, 1), lambda i: (i, 0)),
        out_shape=jax.ShapeDtypeStruct((BATCH, 1), jnp.float32),
    )(e, w0, c0, w1, c1, w2, c2, wo, co)


def kernel(x, emb, W0, b0, g0, be0, W1, b1, g1, be1, W2, b2, g2, be2, Wo, bo):
    idx = x.astype(jnp.int32).reshape(_NW, _BPW)
    flat = _launder(emb.T)
    tab = flat.reshape(EMBED * _GPF, 16)
    e = _gather(tab, idx)

    inv = 1.0 / jnp.sqrt(jnp.float32(1.0 + BN_EPS))
    s0, s1, s2 = g0 * inv, g1 * inv, g2 * inv
    w0 = W0 * s0[None, :]
    c0 = (b0 * s0 + be0).reshape(1, -1)
    w1 = W1 * s1[None, :]
    c1 = (b1 * s1 + be1).reshape(1, -1)
    w2 = W2 * s2[None, :]
    c2 = (b2 * s2 + be2).reshape(1, -1)
    co = bo.reshape(1, 1)

    return _mlp(e, w0, c0, w1, c1, w2, c2, Wo, co)
